# Initial kernel scaffold; baseline (speedup 1.0000x reference)
import jax
import jax.numpy as jnp
from jax.experimental import pallas as pl


def kernel(input, weight):
    raise NotImplementedError("write your pallas kernel here")


# speedup vs baseline: 1.4372x; 1.4372x over previous
import functools

import jax
import jax.numpy as jnp
from jax import lax
from jax.experimental import pallas as pl
from jax.experimental.pallas import tpu as pltpu
from jax.experimental.pallas import tpu_sc as plsc

NUM_EMB = 1000000
DIM = 32
BATCH = 16384
FIELDS = 26
B = BATCH * FIELDS

NC = 2
NS = 16
NW = NC * NS
B_PER_W = B // NW
CHUNK = 128
NCHUNK = B_PER_W // CHUNK
NBUF = 2

_mesh = plsc.VectorSubcoreMesh(core_axis_name="c", subcore_axis_name="s")


@functools.partial(
    pl.kernel,
    mesh=_mesh,
    out_type=jax.ShapeDtypeStruct((B, DIM), jnp.float32),
    compiler_params=pltpu.CompilerParams(use_tc_tiling_on_sc=False),
    scratch_types=[
        pltpu.VMEM((B_PER_W,), jnp.int32),
        pltpu.VMEM((NBUF, CHUNK, DIM), jnp.float32),
        pltpu.SemaphoreType.DMA,
    ],
)
def _gather(idx_hbm, table_hbm, out_hbm, idx_v, rows_v, gsem):
    wid = lax.axis_index("s") * NC + lax.axis_index("c")
    base = wid * B_PER_W
    pltpu.sync_copy(idx_hbm.at[pl.ds(base, B_PER_W)], idx_v)

    def body(j, carry):
        b = j % NBUF
        pltpu.async_copy(
            table_hbm.at[idx_v.at[pl.ds(j * CHUNK, CHUNK)]],
            rows_v.at[b],
            gsem,
        ).wait()
        pltpu.sync_copy(rows_v.at[b], out_hbm.at[pl.ds(base + j * CHUNK, CHUNK)])
        return carry

    lax.fori_loop(0, NCHUNK, body, 0)


def kernel(input, weight):
    idx = input.reshape(-1).astype(jnp.int32)
    out = _gather(idx, weight)
    return out.reshape(BATCH, FIELDS, DIM)


# speedup vs baseline: 1.5656x; 1.0893x over previous
import functools

import jax
import jax.numpy as jnp
from jax import lax
from jax.experimental import pallas as pl
from jax.experimental.pallas import tpu as pltpu
from jax.experimental.pallas import tpu_sc as plsc

NUM_EMB = 1000000
DIM = 32
BATCH = 16384
FIELDS = 26
B = BATCH * FIELDS

NC = 2
NS = 16
NW = NC * NS
B_PER_W = B // NW
CHUNK = 128
G = 8
GROWS = G * CHUNK
NGROUP = B_PER_W // GROWS
NBUF = 2

_mesh = plsc.VectorSubcoreMesh(core_axis_name="c", subcore_axis_name="s")


@functools.partial(
    pl.kernel,
    mesh=_mesh,
    out_type=jax.ShapeDtypeStruct((B, DIM), jnp.float32),
    compiler_params=pltpu.CompilerParams(use_tc_tiling_on_sc=False),
    scratch_types=[
        pltpu.VMEM((B_PER_W,), jnp.int32),
        pltpu.VMEM((NBUF, GROWS, DIM), jnp.float32),
        pltpu.SemaphoreType.DMA,
        pltpu.SemaphoreType.DMA,
    ],
)
def _gather(idx_hbm, table_hbm, out_hbm, idx_v, rows_v, gsem, osem):
    wid = lax.axis_index("s") * NC + lax.axis_index("c")
    base = wid * B_PER_W
    pltpu.sync_copy(idx_hbm.at[pl.ds(base, B_PER_W)], idx_v)

    def fire(g, b):
        for k in range(G):
            pltpu.async_copy(
                table_hbm.at[idx_v.at[pl.ds(g * GROWS + k * CHUNK, CHUNK)]],
                rows_v.at[b, pl.ds(k * CHUNK, CHUNK)],
                gsem,
            )

    def drain_gathers(b):
        for k in range(G):
            pltpu.make_async_copy(
                table_hbm.at[idx_v.at[pl.ds(k * CHUNK, CHUNK)]],
                rows_v.at[b, pl.ds(k * CHUNK, CHUNK)],
                gsem,
            ).wait()

    def out_copy(g, b):
        pltpu.async_copy(
            rows_v.at[b], out_hbm.at[pl.ds(base + g * GROWS, GROWS)], osem
        )

    def wait_out(b):
        pltpu.make_async_copy(
            rows_v.at[b], out_hbm.at[pl.ds(base, GROWS)], osem
        ).wait()

    fire(0, 0)

    def body(g, carry):
        b = g % NBUF
        nb = (g + 1) % NBUF
        drain_gathers(b)

        @pl.when(g + 1 < NGROUP)
        def _():
            @pl.when(g >= 1)
            def _():
                wait_out(nb)

            fire(g + 1, nb)

        out_copy(g, b)
        return carry

    lax.fori_loop(0, NGROUP, body, 0)
    wait_out((NGROUP - 2) % NBUF)
    wait_out((NGROUP - 1) % NBUF)


def kernel(input, weight):
    idx = input.reshape(-1).astype(jnp.int32)
    out = _gather(idx, weight)
    return out.reshape(BATCH, FIELDS, DIM)


# speedup vs baseline: 1.5670x; 1.0009x over previous
import functools

import jax
import jax.numpy as jnp
from jax import lax
from jax.experimental import pallas as pl
from jax.experimental.pallas import tpu as pltpu
from jax.experimental.pallas import tpu_sc as plsc

NUM_EMB = 1000000
DIM = 32
BATCH = 16384
FIELDS = 26
B = BATCH * FIELDS

NC = 2
NS = 16
NW = NC * NS
B_PER_W = B // NW
CHUNK = 1024
G = 1
GROWS = G * CHUNK
NGROUP = B_PER_W // GROWS
NBUF = 2

_mesh = plsc.VectorSubcoreMesh(core_axis_name="c", subcore_axis_name="s")


@functools.partial(
    pl.kernel,
    mesh=_mesh,
    out_type=jax.ShapeDtypeStruct((B, DIM), jnp.float32),
    compiler_params=pltpu.CompilerParams(use_tc_tiling_on_sc=False),
    scratch_types=[
        pltpu.VMEM((B_PER_W,), jnp.int32),
        pltpu.VMEM((NBUF, GROWS, DIM), jnp.float32),
        pltpu.SemaphoreType.DMA,
        pltpu.SemaphoreType.DMA,
    ],
)
def _gather(idx_hbm, table_hbm, out_hbm, idx_v, rows_v, gsem, osem):
    wid = lax.axis_index("s") * NC + lax.axis_index("c")
    base = wid * B_PER_W
    pltpu.sync_copy(idx_hbm.at[pl.ds(base, B_PER_W)], idx_v)

    def fire(g, b):
        for k in range(G):
            pltpu.async_copy(
                table_hbm.at[idx_v.at[pl.ds(g * GROWS + k * CHUNK, CHUNK)]],
                rows_v.at[b, pl.ds(k * CHUNK, CHUNK)],
                gsem,
            )

    def drain_gathers(b):
        for k in range(G):
            pltpu.make_async_copy(
                table_hbm.at[idx_v.at[pl.ds(k * CHUNK, CHUNK)]],
                rows_v.at[b, pl.ds(k * CHUNK, CHUNK)],
                gsem,
            ).wait()

    def out_copy(g, b):
        pltpu.async_copy(
            rows_v.at[b], out_hbm.at[pl.ds(base + g * GROWS, GROWS)], osem
        )

    def wait_out(b):
        pltpu.make_async_copy(
            rows_v.at[b], out_hbm.at[pl.ds(base, GROWS)], osem
        ).wait()

    fire(0, 0)

    def body(g, carry):
        b = g % NBUF
        nb = (g + 1) % NBUF
        drain_gathers(b)

        @pl.when(g + 1 < NGROUP)
        def _():
            @pl.when(g >= 1)
            def _():
                wait_out(nb)

            fire(g + 1, nb)

        out_copy(g, b)
        return carry

    lax.fori_loop(0, NGROUP, body, 0)
    wait_out((NGROUP - 2) % NBUF)
    wait_out((NGROUP - 1) % NBUF)


def kernel(input, weight):
    idx = input.reshape(-1).astype(jnp.int32)
    out = _gather(idx, weight)
    return out.reshape(BATCH, FIELDS, DIM)
